# baseline (speedup 1.0000x reference)
import math

import jax
import jax.numpy as jnp
from jax import lax
from jax.experimental import pallas as pl
from jax.experimental.pallas import tpu as pltpu
from jax.experimental.pallas import tpu_sc as plsc

EMB = 128
SCALE = math.sqrt(float(EMB))

_info = plsc.get_sparse_core_info()
NC = _info.num_cores
NS = _info.num_subcores
LANES = _info.num_lanes
NW = NC * NS

NB0 = 2
NBUF = 2


def _emb_body(seq, idx_hbm, table_hbm, out_hbm,
              idx_v, g0, g1, s0, s1,
              gsem0, gsem1, osem0, osem1, isem):
    nchunk = idx_v.shape[0]
    K = idx_v.shape[1]
    wid = lax.axis_index("s") * NC + lax.axis_index("c")
    b0_base = wid * (nchunk * NB0)

    gbufs = (g0, g1)
    sbufs = (s0, s1)
    gsems = (gsem0, gsem1)
    osems = (osem0, osem1)

    pltpu.async_copy(idx_hbm.at[wid], idx_v, isem).wait()

    for b in range(NBUF):
        pltpu.async_copy(table_hbm.at[idx_v.at[b]], gbufs[b], gsems[b])

    def scale_rows(src, dst):
        def row(r, carry):
            for u in range(NB0):
                for j in range(EMB // LANES):
                    sl = pl.ds(j * LANES, LANES)
                    dst[u, r, sl] = src[u * seq + r, sl] * SCALE
            return carry
        lax.fori_loop(0, seq, row, 0)

    def group(g, carry):
        for b in range(NBUF):
            c = g * NBUF + b
            pltpu.make_async_copy(table_hbm.at[idx_v.at[b]], gbufs[b],
                                  gsems[b]).wait()
            @pl.when(g > 0)
            def _():
                pltpu.make_async_copy(sbufs[b], out_hbm.at[pl.ds(b0_base, NB0)],
                                      osems[b]).wait()
            scale_rows(gbufs[b], sbufs[b])
            pltpu.async_copy(sbufs[b],
                             out_hbm.at[pl.ds(b0_base + c * NB0, NB0)],
                             osems[b])
            @pl.when(c + NBUF < nchunk)
            def _():
                pltpu.async_copy(table_hbm.at[idx_v.at[c + NBUF]],
                                 gbufs[b], gsems[b])
        return carry

    lax.fori_loop(0, nchunk // NBUF, group, 0)

    for b in range(NBUF):
        pltpu.make_async_copy(sbufs[b], out_hbm.at[pl.ds(b0_base, NB0)],
                              osems[b]).wait()


def kernel(tokens, table):
    b0, seq = tokens.shape
    assert b0 % (NW * NB0) == 0
    nchunk = b0 // (NW * NB0)
    K = NB0 * seq

    idx = tokens.reshape(NW, nchunk, K).astype(jnp.int32)

    def body(*args):
        return _emb_body(seq, *args)

    emb = pl.kernel(
        body,
        out_type=jax.ShapeDtypeStruct((b0, seq, EMB), jnp.float32),
        mesh=plsc.VectorSubcoreMesh(core_axis_name="c", subcore_axis_name="s"),
        compiler_params=pltpu.CompilerParams(use_tc_tiling_on_sc=True),
        scratch_types=[
            pltpu.VMEM((nchunk, K), jnp.int32),
            pltpu.VMEM((K, EMB), jnp.float32),
            pltpu.VMEM((K, EMB), jnp.float32),
            pltpu.VMEM((NB0, seq, EMB), jnp.float32),
            pltpu.VMEM((NB0, seq, EMB), jnp.float32),
            pltpu.SemaphoreType.DMA,
            pltpu.SemaphoreType.DMA,
            pltpu.SemaphoreType.DMA,
            pltpu.SemaphoreType.DMA,
            pltpu.SemaphoreType.DMA,
        ],
    )(idx, table)

    return emb


# speedup vs baseline: 1.7828x; 1.7828x over previous
import math

import jax
import jax.numpy as jnp
from jax import lax
from jax.experimental import pallas as pl
from jax.experimental.pallas import tpu as pltpu
from jax.experimental.pallas import tpu_sc as plsc

EMB = 128
SCALE = math.sqrt(float(EMB))

_info = plsc.get_sparse_core_info()
NC = _info.num_cores
NS = _info.num_subcores
LANES = _info.num_lanes
NW = NC * NS

K = 128
NBUF = 2


def _emb_body(idx_hbm, table_hbm, out_hbm,
              idx_v, g0, g1, s0, s1,
              gsem0, gsem1, osem0, osem1, isem):
    nchunk = idx_v.shape[0]
    wid = lax.axis_index("s") * NC + lax.axis_index("c")
    base = wid * (nchunk * K)

    gbufs = (g0, g1)
    sbufs = (s0, s1)
    gsems = (gsem0, gsem1)
    osems = (osem0, osem1)

    pltpu.async_copy(idx_hbm.at[wid], idx_v, isem).wait()

    for b in range(NBUF):
        pltpu.async_copy(table_hbm.at[idx_v.at[b]], gbufs[b], gsems[b])

    def scale_rows(src, dst):
        def row(r, carry):
            for j in range(EMB // LANES):
                sl = pl.ds(j * LANES, LANES)
                dst[r, sl] = src[r, sl] * SCALE
            return carry
        lax.fori_loop(0, K, row, 0)

    def group(g, carry):
        for b in range(NBUF):
            c = g * NBUF + b
            pltpu.make_async_copy(table_hbm.at[idx_v.at[b]], gbufs[b],
                                  gsems[b]).wait()
            @pl.when(g > 0)
            def _():
                pltpu.make_async_copy(sbufs[b], out_hbm.at[pl.ds(base, K)],
                                      osems[b]).wait()
            scale_rows(gbufs[b], sbufs[b])
            pltpu.async_copy(sbufs[b], out_hbm.at[pl.ds(base + c * K, K)],
                             osems[b])
            @pl.when(c + NBUF < nchunk)
            def _():
                pltpu.async_copy(table_hbm.at[idx_v.at[c + NBUF]],
                                 gbufs[b], gsems[b])
        return carry

    lax.fori_loop(0, nchunk // NBUF, group, 0)

    for b in range(NBUF):
        pltpu.make_async_copy(sbufs[b], out_hbm.at[pl.ds(base, K)],
                              osems[b]).wait()


def kernel(tokens, table):
    b0, seq = tokens.shape
    total = b0 * seq
    assert total % (NW * K) == 0
    nchunk = total // (NW * K)

    idx = tokens.astype(jnp.int32).T.reshape(NW, nchunk, K)

    emb = pl.kernel(
        _emb_body,
        out_type=jax.ShapeDtypeStruct((total, EMB), jnp.float32),
        mesh=plsc.VectorSubcoreMesh(core_axis_name="c", subcore_axis_name="s"),
        scratch_types=[
            pltpu.VMEM((nchunk, K), jnp.int32),
            pltpu.VMEM((K, EMB), jnp.float32),
            pltpu.VMEM((K, EMB), jnp.float32),
            pltpu.VMEM((K, EMB), jnp.float32),
            pltpu.VMEM((K, EMB), jnp.float32),
            pltpu.SemaphoreType.DMA,
            pltpu.SemaphoreType.DMA,
            pltpu.SemaphoreType.DMA,
            pltpu.SemaphoreType.DMA,
            pltpu.SemaphoreType.DMA,
        ],
    )(idx, table)

    return emb.reshape(seq, b0, EMB).transpose(1, 0, 2)
